# baseline (speedup 1.0000x reference)
import jax
import jax.numpy as jnp
from jax import lax
from jax.experimental import pallas as pl
from jax.experimental.pallas import tpu as pltpu
from jax.experimental.pallas import tpu_sc as plsc

B, L, D = 16, 2048, 1024
LANES = 16
HALF = D // 2
NVEC = HALF // LANES
CH = 64
BLK = 512
NSTEP = L // BLK
SEQ_PER_SC = B // 2


def _sc_body(xs_hbm, len_hbm, out_hbm, lbuf, buf0, buf1, obuf, sem0, sem1):
    c = lax.axis_index("c")
    s = lax.axis_index("s")
    b = c * SEQ_PER_SC + s // 2
    h = s % 2
    col0 = h * HALF

    pltpu.sync_copy(len_hbm, lbuf)
    lvec = lbuf[...]
    wk = s // 2
    l0 = lvec[0]
    l1 = lvec[SEQ_PER_SC]
    for j in range(1, SEQ_PER_SC):
        l0 = jnp.where(wk == j, lvec[j], l0)
        l1 = jnp.where(wk == j, lvec[SEQ_PER_SC + j], l1)
    l = jnp.where(c == 0, l0, l1)

    base = (l // BLK) * BLK
    n = l - base
    nch = (n + CH - 1) // CH

    def issue(i, buf, sem):
        pltpu.make_async_copy(
            xs_hbm.at[b, pl.ds(base + i * CH, CH), pl.ds(col0, HALF)],
            buf, sem,
        ).start()

    def wait(buf, sem):
        pltpu.make_async_copy(
            xs_hbm.at[b, pl.ds(0, CH), pl.ds(col0, HALF)], buf, sem
        ).wait()

    def accum(i, buf, accs):
        nv = jnp.clip(n - i * CH, 0, CH)

        def row_body(r, a):
            return tuple(
                a[v] + buf[r, pl.ds(v * LANES, LANES)] for v in range(NVEC)
            )

        return lax.fori_loop(0, nv, row_body, accs)

    @pl.when(0 < nch)
    def _():
        issue(0, buf0, sem0)

    @pl.when(1 < nch)
    def _():
        issue(1, buf1, sem1)

    def pair_body(i2, accs):
        ca = 2 * i2
        wait(buf0, sem0)
        accs = accum(ca, buf0, accs)

        @pl.when(ca + 2 < nch)
        def _():
            issue(ca + 2, buf0, sem0)

        @pl.when(ca + 1 < nch)
        def _():
            wait(buf1, sem1)

        accs = accum(ca + 1, buf1, accs)

        @pl.when(ca + 3 < nch)
        def _():
            issue(ca + 3, buf1, sem1)

        return accs

    accs = tuple(jnp.zeros((LANES,), jnp.float32) for _ in range(NVEC))
    accs = lax.fori_loop(0, (nch + 1) // 2, pair_body, accs)

    inv = 1.0 / jnp.full((LANES,), l).astype(jnp.float32)
    for v in range(NVEC):
        obuf[pl.ds(v * LANES, LANES)] = accs[v] * inv
    pltpu.sync_copy(obuf, out_hbm.at[b, pl.ds(col0, HALF)])


def _tc_body(len_ref, b_ref, i_ref, x_hbm, o_ref, k_ref):
    k_ref[0] = 0
    o_ref[...] = jnp.zeros_like(o_ref)
    nblk = len_ref[0] // BLK
    for j in range(1, B):
        nblk = nblk + len_ref[j] // BLK

    w8 = jnp.full((8, BLK), 0.125, dtype=jnp.float32)

    def inner(x_blk):
        k = k_ref[0]
        bb = b_ref[k]
        o_ref[pl.ds(bb, 1)] += jnp.dot(
            w8, x_blk[0], preferred_element_type=jnp.float32
        )[None]
        k_ref[0] = k + 1

    @pl.when(nblk > 0)
    def _():
        pltpu.emit_pipeline(
            inner,
            grid=(nblk,),
            in_specs=[
                pl.BlockSpec((1, BLK, D), lambda k: (b_ref[k], i_ref[k], 0))
            ],
        )(x_hbm)


@jax.jit
def _mean_pool(xs, lens):
    sc_kern = pl.kernel(
        _sc_body,
        out_type=jax.ShapeDtypeStruct((B, D), jnp.float32),
        mesh=plsc.VectorSubcoreMesh(core_axis_name="c", subcore_axis_name="s"),
        scratch_types=[
            pltpu.VMEM((LANES,), jnp.int32),
            pltpu.VMEM((CH, HALF), jnp.float32),
            pltpu.VMEM((CH, HALF), jnp.float32),
            pltpu.VMEM((HALF,), jnp.float32),
            pltpu.SemaphoreType.DMA,
            pltpu.SemaphoreType.DMA,
        ],
    )
    sc_out = sc_kern(xs, lens)

    nfull = lens // BLK
    cum = jnp.cumsum(nfull)
    karr = jnp.arange(B * NSTEP, dtype=jnp.int32)
    b_arr = jnp.clip(jnp.searchsorted(cum, karr, side="right"),
                     0, B - 1).astype(jnp.int32)
    i_arr = jnp.clip(karr - (cum - nfull)[b_arr], 0, NSTEP - 1)

    tc_out = pl.pallas_call(
        _tc_body,
        grid_spec=pltpu.PrefetchScalarGridSpec(
            num_scalar_prefetch=3,
            grid=(1,),
            in_specs=[pl.BlockSpec(memory_space=pl.ANY)],
            out_specs=pl.BlockSpec((B, 8, D), lambda i, *_: (0, 0, 0)),
            scratch_shapes=[pltpu.SMEM((1,), jnp.int32)],
        ),
        out_shape=jax.ShapeDtypeStruct((B, 8, D), jnp.float32),
    )(lens, b_arr, i_arr, xs)

    inv = 1.0 / lens.astype(jnp.float32)
    return sc_out + jnp.sum(tc_out, axis=1) * inv[:, None]


def kernel(xs, xs_len):
    return _mean_pool(xs, xs_len.astype(jnp.int32))


# speedup vs baseline: 1.0413x; 1.0413x over previous
import jax
import jax.numpy as jnp
from jax import lax
from jax.experimental import pallas as pl
from jax.experimental.pallas import tpu as pltpu
from jax.experimental.pallas import tpu_sc as plsc

B, L, D = 16, 2048, 1024
LANES = 16
HALF = D // 2
NVEC = HALF // LANES
CH = 64
BLK = 512
NSTEP = L // BLK
SEQ_PER_SC = B // 2


def _sc_body(xs_hbm, len_hbm, out_hbm, lbuf, buf0, buf1, obuf, sem0, sem1):
    c = lax.axis_index("c")
    s = lax.axis_index("s")
    b = c * SEQ_PER_SC + s // 2
    h = s % 2
    col0 = h * HALF

    pltpu.sync_copy(len_hbm, lbuf)
    lvec = lbuf[...]
    wk = s // 2
    l0 = lvec[0]
    l1 = lvec[SEQ_PER_SC]
    for j in range(1, SEQ_PER_SC):
        l0 = jnp.where(wk == j, lvec[j], l0)
        l1 = jnp.where(wk == j, lvec[SEQ_PER_SC + j], l1)
    l = jnp.where(c == 0, l0, l1)

    base = (l // BLK) * BLK
    n = l - base
    nch = (n + CH - 1) // CH

    def issue(i, buf, sem):
        pltpu.make_async_copy(
            xs_hbm.at[b, pl.ds(base + i * CH, CH), pl.ds(col0, HALF)],
            buf, sem,
        ).start()

    def wait(buf, sem):
        pltpu.make_async_copy(
            xs_hbm.at[b, pl.ds(0, CH), pl.ds(col0, HALF)], buf, sem
        ).wait()

    def accum(i, buf, accs):
        nv = jnp.clip(n - i * CH, 0, CH)

        def row_body(r, a):
            return tuple(
                a[v] + buf[r, pl.ds(v * LANES, LANES)] for v in range(NVEC)
            )

        return lax.fori_loop(0, nv, row_body, accs)

    @pl.when(0 < nch)
    def _():
        issue(0, buf0, sem0)

    @pl.when(1 < nch)
    def _():
        issue(1, buf1, sem1)

    def pair_body(i2, accs):
        ca = 2 * i2
        wait(buf0, sem0)
        accs = accum(ca, buf0, accs)

        @pl.when(ca + 2 < nch)
        def _():
            issue(ca + 2, buf0, sem0)

        @pl.when(ca + 1 < nch)
        def _():
            wait(buf1, sem1)

        accs = accum(ca + 1, buf1, accs)

        @pl.when(ca + 3 < nch)
        def _():
            issue(ca + 3, buf1, sem1)

        return accs

    accs = tuple(jnp.zeros((LANES,), jnp.float32) for _ in range(NVEC))
    accs = lax.fori_loop(0, (nch + 1) // 2, pair_body, accs)

    inv = 1.0 / jnp.full((LANES,), l).astype(jnp.float32)
    for v in range(NVEC):
        obuf[pl.ds(v * LANES, LANES)] = accs[v] * inv
    pltpu.sync_copy(obuf, out_hbm.at[b, pl.ds(col0, HALF)])


def _tc_body(len_ref, b_ref, i_ref, x_hbm, o_ref, k_ref):
    k_ref[0] = 0
    o_ref[...] = jnp.zeros_like(o_ref)
    nblk = len_ref[0] // BLK
    for j in range(1, B):
        nblk = nblk + len_ref[j] // BLK

    def inner(x_blk):
        k = k_ref[0]
        bb = b_ref[k]
        o_ref[pl.ds(bb, 1)] += jnp.sum(
            x_blk[0].reshape(BLK // 8, 8, D), axis=0
        )[None]
        k_ref[0] = k + 1

    @pl.when(nblk > 0)
    def _():
        pltpu.emit_pipeline(
            inner,
            grid=(nblk,),
            in_specs=[
                pl.BlockSpec((1, BLK, D), lambda k: (b_ref[k], i_ref[k], 0))
            ],
        )(x_hbm)


@jax.jit
def _mean_pool(xs, lens):
    sc_kern = pl.kernel(
        _sc_body,
        out_type=jax.ShapeDtypeStruct((B, D), jnp.float32),
        mesh=plsc.VectorSubcoreMesh(core_axis_name="c", subcore_axis_name="s"),
        scratch_types=[
            pltpu.VMEM((LANES,), jnp.int32),
            pltpu.VMEM((CH, HALF), jnp.float32),
            pltpu.VMEM((CH, HALF), jnp.float32),
            pltpu.VMEM((HALF,), jnp.float32),
            pltpu.SemaphoreType.DMA,
            pltpu.SemaphoreType.DMA,
        ],
    )
    sc_out = sc_kern(xs, lens)

    nfull = lens // BLK
    cum = jnp.cumsum(nfull)
    karr = jnp.arange(B * NSTEP, dtype=jnp.int32)
    b_arr = jnp.clip(jnp.searchsorted(cum, karr, side="right"),
                     0, B - 1).astype(jnp.int32)
    i_arr = jnp.clip(karr - (cum - nfull)[b_arr], 0, NSTEP - 1)

    tc_out = pl.pallas_call(
        _tc_body,
        grid_spec=pltpu.PrefetchScalarGridSpec(
            num_scalar_prefetch=3,
            grid=(1,),
            in_specs=[pl.BlockSpec(memory_space=pl.ANY)],
            out_specs=pl.BlockSpec((B, 8, D), lambda i, *_: (0, 0, 0)),
            scratch_shapes=[pltpu.SMEM((1,), jnp.int32)],
        ),
        out_shape=jax.ShapeDtypeStruct((B, 8, D), jnp.float32),
    )(lens, b_arr, i_arr, xs)

    inv = 1.0 / lens.astype(jnp.float32)
    return sc_out + jnp.sum(tc_out, axis=1) * inv[:, None]


def kernel(xs, xs_len):
    return _mean_pool(xs, xs_len.astype(jnp.int32))
